# baseline (speedup 1.0000x reference)
import math
import functools

import jax
import jax.numpy as jnp
from jax import lax
from jax.experimental import pallas as pl
from jax.experimental.pallas import tpu as pltpu
from jax.experimental.pallas import tpu_sc as plsc

_NC = 2
_NS = 16
_NW = _NC * _NS
_L = 16


def _sc_lookup(coords_flat, row_embed, col_embed, *, grid_size, num_emb, dh):
  total = coords_flat.shape[0]
  points = total // 2
  ppw = points // _NW
  cpw = 2 * ppw
  chunk = 32
  nb = 4
  n_chunks = ppw // chunk
  mchunk = total // _NS
  nv = min(grid_size + 1, num_emb)
  nvs = (nv + 7) // 8 * 8
  nvp = 3 * _L
  rpt = (nv + _NS - 1) // _NS

  mesh = plsc.VectorSubcoreMesh(
      core_axis_name="c", subcore_axis_name="s",
      num_cores=_NC, num_subcores=_NS)

  @functools.partial(
      pl.kernel,
      out_type=(
          jax.ShapeDtypeStruct((points, 2 * dh), jnp.float32),
          jax.ShapeDtypeStruct((nv * nvs, 2 * dh), jnp.float32),
      ),
      mesh=mesh,
      compiler_params=pltpu.CompilerParams(needs_layout_passes=False),
      scratch_types=[
          pltpu.VMEM((mchunk,), jnp.float32),
          pltpu.VMEM((cpw,), jnp.float32),
          pltpu.VMEM((ppw,), jnp.int32),
          pltpu.VMEM((nvp,), jnp.int32),
          pltpu.VMEM((nvp,), jnp.int32),
          pltpu.VMEM((_L,), jnp.float32),
          pltpu.VMEM((_NS, _L), jnp.float32),
          pltpu.VMEM_SHARED((_NS, _L), jnp.float32),
      ] + [pltpu.VMEM((chunk, 2 * dh), jnp.float32)] * nb
        + [pltpu.SemaphoreType.DMA] * (2 * nb),
  )
  def body(coords_hbm, row_hbm, col_hbm, out_hbm, comb_hbm,
           mbuf, cbuf, cidx, ibr, ibc, pbuf, lmax, shmax, *bufs_sems):
    kbufs = bufs_sems[:nb]
    gsems = bufs_sems[nb:2 * nb]
    wsems = bufs_sems[2 * nb:3 * nb]
    cid = lax.axis_index("c")
    sid = lax.axis_index("s")
    wid = sid * _NC + cid

    pltpu.sync_copy(coords_hbm.at[pl.ds(sid * mchunk, mchunk)], mbuf)

    def red(i, accs):
      a0, a1, a2, a3 = accs
      base = i * (4 * _L)
      a0 = jnp.maximum(a0, mbuf[pl.ds(base, _L)])
      a1 = jnp.maximum(a1, mbuf[pl.ds(base + _L, _L)])
      a2 = jnp.maximum(a2, mbuf[pl.ds(base + 2 * _L, _L)])
      a3 = jnp.maximum(a3, mbuf[pl.ds(base + 3 * _L, _L)])
      return (a0, a1, a2, a3)

    neg = jnp.full((_L,), -jnp.inf, dtype=jnp.float32)
    a0, a1, a2, a3 = lax.fori_loop(0, mchunk // (4 * _L), red,
                                   (neg, neg, neg, neg))
    pbuf[pl.ds(0, _L)] = jnp.maximum(jnp.maximum(a0, a1),
                                     jnp.maximum(a2, a3))
    pltpu.sync_copy(pbuf, shmax.at[sid])

    for i in range(nvp // _L):
      ibc[pl.ds(i * _L, _L)] = lax.iota(jnp.int32, _L) + i * _L

    lo = chunk
    hi = nvs - chunk
    for rr in range(rpt):
      r = sid * rpt + rr

      @pl.when(r < nv)
      def _():
        for i in range(nvp // _L):
          ibr[pl.ds(i * _L, _L)] = jnp.full((_L,), r, dtype=jnp.int32)
        cps = (
            pltpu.make_async_copy(
                row_hbm.at[ibr.at[pl.ds(0, lo)]],
                kbufs[0].at[:, pl.ds(0, dh)], gsems[0]),
            pltpu.make_async_copy(
                col_hbm.at[ibc.at[pl.ds(0, lo)]],
                kbufs[0].at[:, pl.ds(dh, dh)], gsems[0]),
            pltpu.make_async_copy(
                row_hbm.at[ibr.at[pl.ds(lo, hi)]],
                kbufs[1].at[pl.ds(0, hi), pl.ds(0, dh)], gsems[1]),
            pltpu.make_async_copy(
                col_hbm.at[ibc.at[pl.ds(lo, hi)]],
                kbufs[1].at[pl.ds(0, hi), pl.ds(dh, dh)], gsems[1]),
        )
        for cp in cps:
          cp.start()
        for cp in cps:
          cp.wait()
        pltpu.sync_copy(kbufs[0], comb_hbm.at[pl.ds(r * nvs, lo)])
        pltpu.sync_copy(kbufs[1].at[pl.ds(0, hi)],
                        comb_hbm.at[pl.ds(r * nvs + lo, hi)])

    pltpu.sync_copy(coords_hbm.at[pl.ds(wid * cpw, cpw)], cbuf)

    plsc.subcore_barrier()

    pltpu.sync_copy(shmax, lmax)
    acc = lmax[0]
    for i in range(1, _NS):
      acc = jnp.maximum(acc, lmax[i])
    iota = lax.iota(jnp.int32, _L)
    for s in (1, 2, 4, 8):
      acc = jnp.maximum(acc, acc.at[iota ^ s].get(mode="promise_in_bounds"))
    max_coord = acc

    gs = jnp.float32(grid_size)

    def idx_group(g):
      base = g * (2 * _L)
      xi = base + 2 * iota
      x = plsc.load_gather(cbuf, [xi])
      y = plsc.load_gather(cbuf, [xi + 1])
      r = jnp.clip((y / max_coord * gs).astype(jnp.int32), 0, nv - 1)
      c = jnp.clip((x / max_coord * gs).astype(jnp.int32), 0, nv - 1)
      cidx[pl.ds(g * _L, _L)] = r * nvs + c

    def g_copies(k, b):
      p0 = k * chunk
      return (
          pltpu.make_async_copy(
              comb_hbm.at[cidx.at[pl.ds(p0, chunk)]], kbufs[b], gsems[b]),
      )

    def w_copies(k, b):
      o0 = wid * ppw + k * chunk
      return (
          pltpu.make_async_copy(
              kbufs[b], out_hbm.at[pl.ds(o0, chunk)], wsems[b]),
      )

    def issue(copies):
      for c in copies:
        c.start()

    def drain(copies):
      for c in copies:
        c.wait()

    head_groups = nb * chunk // _L
    for g in range(head_groups):
      idx_group(g)
    for b in range(nb):
      issue(g_copies(b, b))

    def idx_step(g, _):
      idx_group(g)
      return 0

    lax.fori_loop(head_groups, ppw // _L, idx_step, 0)

    n_super = n_chunks // nb

    def pipe_step(jj, _):
      for b in range(nb):
        k = jj * nb + b
        drain(g_copies(k, b))
        issue(w_copies(k, b))

        @pl.when(jj < n_super - 1)
        def _():
          drain(w_copies(k, b))
          issue(g_copies(k + nb, b))
      return 0

    lax.fori_loop(0, n_super, pipe_step, 0)
    for b in range(nb):
      drain(w_copies(n_chunks - nb + b, b))

  return body(coords_flat, row_embed, col_embed)


def kernel(patch_coords, row_embed, col_embed):
  b, n, _ = patch_coords.shape
  num_emb, dh = row_embed.shape
  grid_size = int(math.sqrt(n)) + 1
  points = b * n
  assert points % (_NW * 128) == 0

  coords_flat = jnp.reshape(patch_coords, (2 * points,))
  out, _unused_comb = _sc_lookup(coords_flat, row_embed, col_embed,
                                 grid_size=grid_size, num_emb=num_emb, dh=dh)
  return jnp.reshape(out, (b, n, 2 * dh))


# speedup vs baseline: 1.1978x; 1.1978x over previous
import math
import functools

import jax
import jax.numpy as jnp
from jax import lax
from jax.experimental import pallas as pl
from jax.experimental.pallas import tpu as pltpu
from jax.experimental.pallas import tpu_sc as plsc

_NC = 2
_NS = 16
_NW = _NC * _NS
_L = 16


def _sc_lookup(coords_flat, row_embed, col_embed, *, grid_size, num_emb, dh):
  total = coords_flat.shape[0]
  points = total // 2
  ppw = points // _NW
  cpw = 2 * ppw
  chunk = 32
  nb = 4
  n_chunks = ppw // chunk
  mchunk = 8192
  n_max_chunks = total // mchunk
  nv = min(grid_size + 1, num_emb)
  nvs = (nv + 7) // 8 * 8
  nvp = 3 * _L
  rpt = (nv + _NS - 1) // _NS

  mesh = plsc.VectorSubcoreMesh(
      core_axis_name="c", subcore_axis_name="s",
      num_cores=_NC, num_subcores=_NS)

  @functools.partial(
      pl.kernel,
      out_type=(
          jax.ShapeDtypeStruct((points, 2 * dh), jnp.float32),
          jax.ShapeDtypeStruct((nv * nvs, 2 * dh), jnp.float32),
      ),
      mesh=mesh,
      compiler_params=pltpu.CompilerParams(needs_layout_passes=False),
      scratch_types=[
          pltpu.VMEM((mchunk,), jnp.float32),
          pltpu.VMEM((cpw,), jnp.float32),
          pltpu.VMEM((ppw,), jnp.int32),
          pltpu.VMEM((nvp,), jnp.int32),
          pltpu.VMEM((nvp,), jnp.int32),
      ] + [pltpu.VMEM((chunk, 2 * dh), jnp.float32)] * nb
        + [pltpu.SemaphoreType.DMA] * (2 * nb),
  )
  def body(coords_hbm, row_hbm, col_hbm, out_hbm, comb_hbm,
           mbuf, cbuf, cidx, ibr, ibc, *bufs_sems):
    kbufs = bufs_sems[:nb]
    gsems = bufs_sems[nb:2 * nb]
    wsems = bufs_sems[2 * nb:3 * nb]
    cid = lax.axis_index("c")
    sid = lax.axis_index("s")
    wid = sid * _NC + cid

    for i in range(nvp // _L):
      ibc[pl.ds(i * _L, _L)] = lax.iota(jnp.int32, _L) + i * _L

    lo = chunk
    hi = nvs - chunk
    for rr in range(rpt):
      r = sid * rpt + rr

      @pl.when(r < nv)
      def _():
        for i in range(nvp // _L):
          ibr[pl.ds(i * _L, _L)] = jnp.full((_L,), r, dtype=jnp.int32)
        cps = (
            pltpu.make_async_copy(
                row_hbm.at[ibr.at[pl.ds(0, lo)]],
                kbufs[0].at[:, pl.ds(0, dh)], gsems[0]),
            pltpu.make_async_copy(
                col_hbm.at[ibc.at[pl.ds(0, lo)]],
                kbufs[0].at[:, pl.ds(dh, dh)], gsems[0]),
            pltpu.make_async_copy(
                row_hbm.at[ibr.at[pl.ds(lo, hi)]],
                kbufs[1].at[pl.ds(0, hi), pl.ds(0, dh)], gsems[1]),
            pltpu.make_async_copy(
                col_hbm.at[ibc.at[pl.ds(lo, hi)]],
                kbufs[1].at[pl.ds(0, hi), pl.ds(dh, dh)], gsems[1]),
        )
        for cp in cps:
          cp.start()
        for cp in cps:
          cp.wait()
        pltpu.sync_copy(kbufs[0], comb_hbm.at[pl.ds(r * nvs, lo)])
        pltpu.sync_copy(kbufs[1].at[pl.ds(0, hi)],
                        comb_hbm.at[pl.ds(r * nvs + lo, hi)])

    plsc.subcore_barrier()

    def max_step(j, accs):
      pltpu.sync_copy(coords_hbm.at[pl.ds(j * mchunk, mchunk)], mbuf)
      def red(i, accs):
        a0, a1, a2, a3 = accs
        base = i * (4 * _L)
        a0 = jnp.maximum(a0, mbuf[pl.ds(base, _L)])
        a1 = jnp.maximum(a1, mbuf[pl.ds(base + _L, _L)])
        a2 = jnp.maximum(a2, mbuf[pl.ds(base + 2 * _L, _L)])
        a3 = jnp.maximum(a3, mbuf[pl.ds(base + 3 * _L, _L)])
        return (a0, a1, a2, a3)
      return lax.fori_loop(0, mchunk // (4 * _L), red, accs)

    neg = jnp.full((_L,), -jnp.inf, dtype=jnp.float32)
    a0, a1, a2, a3 = lax.fori_loop(0, n_max_chunks, max_step,
                                   (neg, neg, neg, neg))
    acc = jnp.maximum(jnp.maximum(a0, a1), jnp.maximum(a2, a3))
    iota = lax.iota(jnp.int32, _L)
    for s in (1, 2, 4, 8):
      acc = jnp.maximum(acc, acc.at[iota ^ s].get(mode="promise_in_bounds"))
    max_coord = acc

    pltpu.sync_copy(coords_hbm.at[pl.ds(wid * cpw, cpw)], cbuf)

    gs = jnp.float32(grid_size)

    def idx_group(g):
      base = g * (2 * _L)
      xi = base + 2 * iota
      x = plsc.load_gather(cbuf, [xi])
      y = plsc.load_gather(cbuf, [xi + 1])
      r = jnp.clip((y / max_coord * gs).astype(jnp.int32), 0, nv - 1)
      c = jnp.clip((x / max_coord * gs).astype(jnp.int32), 0, nv - 1)
      cidx[pl.ds(g * _L, _L)] = r * nvs + c

    def g_copies(k, b):
      p0 = k * chunk
      return (
          pltpu.make_async_copy(
              comb_hbm.at[cidx.at[pl.ds(p0, chunk)]], kbufs[b], gsems[b]),
      )

    def w_copies(k, b):
      o0 = wid * ppw + k * chunk
      return (
          pltpu.make_async_copy(
              kbufs[b], out_hbm.at[pl.ds(o0, chunk)], wsems[b]),
      )

    def issue(copies):
      for c in copies:
        c.start()

    def drain(copies):
      for c in copies:
        c.wait()

    head_groups = nb * chunk // _L
    for g in range(head_groups):
      idx_group(g)
    for b in range(nb):
      issue(g_copies(b, b))

    def idx_step(g, _):
      idx_group(g)
      return 0

    lax.fori_loop(head_groups, ppw // _L, idx_step, 0)

    n_super = n_chunks // nb

    def pipe_step(jj, _):
      for b in range(nb):
        k = jj * nb + b
        drain(g_copies(k, b))
        issue(w_copies(k, b))

        @pl.when(jj < n_super - 1)
        def _():
          drain(w_copies(k, b))
          issue(g_copies(k + nb, b))
      return 0

    lax.fori_loop(0, n_super, pipe_step, 0)
    for b in range(nb):
      drain(w_copies(n_chunks - nb + b, b))

  return body(coords_flat, row_embed, col_embed)


def kernel(patch_coords, row_embed, col_embed):
  b, n, _ = patch_coords.shape
  num_emb, dh = row_embed.shape
  grid_size = int(math.sqrt(n)) + 1
  points = b * n
  assert points % (_NW * 128) == 0

  coords_flat = jnp.reshape(patch_coords, (2 * points,))
  out, _unused_comb = _sc_lookup(coords_flat, row_embed, col_embed,
                                 grid_size=grid_size, num_emb=num_emb, dh=dh)
  return jnp.reshape(out, (b, n, 2 * dh))


# speedup vs baseline: 1.3148x; 1.0977x over previous
import math
import functools

import jax
import jax.numpy as jnp
from jax import lax
from jax.experimental import pallas as pl
from jax.experimental.pallas import tpu as pltpu
from jax.experimental.pallas import tpu_sc as plsc

_NC = 2
_NS = 16
_NW = _NC * _NS
_L = 16


def _sc_lookup(coords_flat, row_embed, col_embed, *, grid_size, num_emb, dh):
  total = coords_flat.shape[0]
  points = total // 2
  ppw = points // _NW
  cpw = 2 * ppw
  chunk = 32
  nb = 4
  n_chunks = ppw // chunk
  mchunk = total // _NS
  nv = min(grid_size + 1, num_emb)
  nvs = (nv + 7) // 8 * 8
  nvp = 3 * _L
  rpt = (nv + _NS - 1) // _NS

  mesh = plsc.VectorSubcoreMesh(
      core_axis_name="c", subcore_axis_name="s",
      num_cores=_NC, num_subcores=_NS)

  @functools.partial(
      pl.kernel,
      out_type=(
          jax.ShapeDtypeStruct((points, 2 * dh), jnp.float32),
          jax.ShapeDtypeStruct((nv * nvs, 2 * dh), jnp.float32),
          jax.ShapeDtypeStruct((_NW * _L,), jnp.float32),
      ),
      mesh=mesh,
      compiler_params=pltpu.CompilerParams(needs_layout_passes=False),
      scratch_types=[
          pltpu.VMEM((mchunk,), jnp.float32),
          pltpu.VMEM((cpw,), jnp.float32),
          pltpu.VMEM((ppw,), jnp.int32),
          pltpu.VMEM((nvp,), jnp.int32),
          pltpu.VMEM((nvp,), jnp.int32),
          pltpu.VMEM((_L,), jnp.float32),
          pltpu.VMEM((_NS * _L,), jnp.float32),
      ] + [pltpu.VMEM((chunk, 2 * dh), jnp.float32)] * nb
        + [pltpu.SemaphoreType.DMA] * (2 * nb),
  )
  def body(coords_hbm, row_hbm, col_hbm, out_hbm, comb_hbm, pmax_hbm,
           mbuf, cbuf, cidx, ibr, ibc, pbuf, lbuf, *bufs_sems):
    kbufs = bufs_sems[:nb]
    gsems = bufs_sems[nb:2 * nb]
    wsems = bufs_sems[2 * nb:3 * nb]
    cid = lax.axis_index("c")
    sid = lax.axis_index("s")
    wid = sid * _NC + cid

    pltpu.sync_copy(coords_hbm.at[pl.ds(sid * mchunk, mchunk)], mbuf)

    def red(i, accs):
      a0, a1, a2, a3 = accs
      base = i * (4 * _L)
      a0 = jnp.maximum(a0, mbuf[pl.ds(base, _L)])
      a1 = jnp.maximum(a1, mbuf[pl.ds(base + _L, _L)])
      a2 = jnp.maximum(a2, mbuf[pl.ds(base + 2 * _L, _L)])
      a3 = jnp.maximum(a3, mbuf[pl.ds(base + 3 * _L, _L)])
      return (a0, a1, a2, a3)

    neg = jnp.full((_L,), -jnp.inf, dtype=jnp.float32)
    a0, a1, a2, a3 = lax.fori_loop(0, mchunk // (4 * _L), red,
                                   (neg, neg, neg, neg))
    pbuf[pl.ds(0, _L)] = jnp.maximum(jnp.maximum(a0, a1),
                                     jnp.maximum(a2, a3))
    pltpu.sync_copy(pbuf, pmax_hbm.at[pl.ds((cid * _NS + sid) * _L, _L)])

    pltpu.sync_copy(coords_hbm.at[pl.ds(wid * cpw, cpw)], cbuf)

    for i in range(nvp // _L):
      ibc[pl.ds(i * _L, _L)] = lax.iota(jnp.int32, _L) + i * _L

    lo = chunk
    hi = nvs - chunk
    for rr in range(rpt):
      r = sid * rpt + rr

      @pl.when(r < nv)
      def _():
        for i in range(nvp // _L):
          ibr[pl.ds(i * _L, _L)] = jnp.full((_L,), r, dtype=jnp.int32)
        cps = (
            pltpu.make_async_copy(
                row_hbm.at[ibr.at[pl.ds(0, lo)]],
                kbufs[0].at[:, pl.ds(0, dh)], gsems[0]),
            pltpu.make_async_copy(
                col_hbm.at[ibc.at[pl.ds(0, lo)]],
                kbufs[0].at[:, pl.ds(dh, dh)], gsems[0]),
            pltpu.make_async_copy(
                row_hbm.at[ibr.at[pl.ds(lo, hi)]],
                kbufs[1].at[pl.ds(0, hi), pl.ds(0, dh)], gsems[1]),
            pltpu.make_async_copy(
                col_hbm.at[ibc.at[pl.ds(lo, hi)]],
                kbufs[1].at[pl.ds(0, hi), pl.ds(dh, dh)], gsems[1]),
        )
        for cp in cps:
          cp.start()
        for cp in cps:
          cp.wait()
        pltpu.sync_copy(kbufs[0], comb_hbm.at[pl.ds(r * nvs, lo)])
        pltpu.sync_copy(kbufs[1].at[pl.ds(0, hi)],
                        comb_hbm.at[pl.ds(r * nvs + lo, hi)])

    plsc.subcore_barrier()

    pltpu.sync_copy(pmax_hbm.at[pl.ds(cid * _NS * _L, _NS * _L)], lbuf)
    acc = lbuf[pl.ds(0, _L)]
    for i in range(1, _NS):
      acc = jnp.maximum(acc, lbuf[pl.ds(i * _L, _L)])
    iota = lax.iota(jnp.int32, _L)
    for s in (1, 2, 4, 8):
      acc = jnp.maximum(acc, acc.at[iota ^ s].get(mode="promise_in_bounds"))
    max_coord = acc

    gs = jnp.float32(grid_size)

    def idx_group(g):
      base = g * (2 * _L)
      xi = base + 2 * iota
      x = plsc.load_gather(cbuf, [xi])
      y = plsc.load_gather(cbuf, [xi + 1])
      r = jnp.clip((y / max_coord * gs).astype(jnp.int32), 0, nv - 1)
      c = jnp.clip((x / max_coord * gs).astype(jnp.int32), 0, nv - 1)
      cidx[pl.ds(g * _L, _L)] = r * nvs + c

    def g_copies(k, b):
      p0 = k * chunk
      return (
          pltpu.make_async_copy(
              comb_hbm.at[cidx.at[pl.ds(p0, chunk)]], kbufs[b], gsems[b]),
      )

    def w_copies(k, b):
      o0 = wid * ppw + k * chunk
      return (
          pltpu.make_async_copy(
              kbufs[b], out_hbm.at[pl.ds(o0, chunk)], wsems[b]),
      )

    def issue(copies):
      for c in copies:
        c.start()

    def drain(copies):
      for c in copies:
        c.wait()

    gpc = chunk // _L

    def idx_chunk(k):
      for gg in range(gpc):
        idx_group(k * gpc + gg)

    for b in range(nb):
      idx_chunk(b)
      issue(g_copies(b, b))

    n_super = n_chunks // nb

    def pipe_step(jj, _):
      for b in range(nb):
        k = jj * nb + b

        @pl.when(jj < n_super - 1)
        def _():
          idx_chunk(k + nb)
        drain(g_copies(k, b))
        issue(w_copies(k, b))

        @pl.when(jj < n_super - 1)
        def _():
          drain(w_copies(k, b))
          issue(g_copies(k + nb, b))
      return 0

    lax.fori_loop(0, n_super, pipe_step, 0)
    for b in range(nb):
      drain(w_copies(n_chunks - nb + b, b))

  return body(coords_flat, row_embed, col_embed)


def kernel(patch_coords, row_embed, col_embed):
  b, n, _ = patch_coords.shape
  num_emb, dh = row_embed.shape
  grid_size = int(math.sqrt(n)) + 1
  points = b * n
  assert points % (_NW * 128) == 0

  coords_flat = jnp.reshape(patch_coords, (2 * points,))
  out, _unused_comb, _unused_pmax = _sc_lookup(
      coords_flat, row_embed, col_embed,
      grid_size=grid_size, num_emb=num_emb, dh=dh)
  return jnp.reshape(out, (b, n, 2 * dh))
